# baseline (speedup 1.0000x reference)
import functools

import jax
import jax.numpy as jnp
from jax.experimental import pallas as pl


def _rvq_block(z_ref, cb_ref, cb2_ref, zq_ref, idx_ref, *, n_cb, K, cd):
    rows = z_ref.shape[0]
    iota = jax.lax.broadcasted_iota(jnp.int32, (rows, K), 1)
    carry = jnp.zeros((rows, cd), dtype=jnp.float32)
    idx_cols = []
    for i in range(n_cb):
        res = z_ref[:, i * cd:(i + 1) * cd] + carry
        cb = cb_ref[i]
        scores = cb2_ref[i:i + 1, :] - 2.0 * jax.lax.dot_general(
            res.astype(jnp.bfloat16), cb.astype(jnp.bfloat16),
            (((1,), (1,)), ((), ())),
            preferred_element_type=jnp.float32)
        m = jnp.min(scores, axis=1, keepdims=True)
        idx = jnp.min(jnp.where(scores == m, iota, K), axis=1, keepdims=True)
        onehot = (iota == idx).astype(jnp.float32)
        q = jax.lax.dot_general(
            onehot, cb, (((1,), (0,)), ((), ())),
            preferred_element_type=jnp.float32,
            precision=jax.lax.Precision.HIGHEST)
        zq_ref[:, i * cd:(i + 1) * cd] = q
        idx_cols.append(idx)
        if i < n_cb - 1:
            carry = res - q
    idx_ref[...] = jnp.concatenate(idx_cols, axis=1)


@functools.partial(jax.jit, static_argnames=())
def kernel(z, codebooks):
    B, T, D = z.shape
    n_cb, K, cd = codebooks.shape
    rows = B * T
    R = 512
    zf = z.reshape(rows, D)
    cb2 = jnp.sum(codebooks * codebooks, axis=-1)

    zq_flat, idx_flat = pl.pallas_call(
        functools.partial(_rvq_block, n_cb=n_cb, K=K, cd=cd),
        grid=(rows // R,),
        in_specs=[
            pl.BlockSpec((R, D), lambda b: (b, 0)),
            pl.BlockSpec((n_cb, K, cd), lambda b: (0, 0, 0)),
            pl.BlockSpec((n_cb, K), lambda b: (0, 0)),
        ],
        out_specs=[
            pl.BlockSpec((R, D), lambda b: (b, 0)),
            pl.BlockSpec((R, n_cb), lambda b: (b, 0)),
        ],
        out_shape=[
            jax.ShapeDtypeStruct((rows, D), jnp.float32),
            jax.ShapeDtypeStruct((rows, n_cb), jnp.int32),
        ],
    )(zf, codebooks, cb2)

    z_q = zq_flat.reshape(B, T, D)
    indices = idx_flat.reshape(B, T, n_cb).transpose(0, 2, 1)
    return (z_q, indices)


# speedup vs baseline: 2.8591x; 2.8591x over previous
import functools

import jax
import jax.numpy as jnp
from jax.experimental import pallas as pl


def _rvq_block(zt_ref, cbbf_ref, cb2_ref, hi_ref, mid_ref, lo_ref,
               zqt_ref, idx_ref, *, n_cb, K, cd):
    R = zt_ref.shape[1]
    iota0 = jax.lax.broadcasted_iota(jnp.int32, (K, R), 0)
    carry = jnp.zeros((cd, R), dtype=jnp.float32)
    for i in range(n_cb):
        res = zt_ref[i * cd:(i + 1) * cd, :] + carry
        mm = jax.lax.dot_general(
            cbbf_ref[i], res.astype(jnp.bfloat16),
            (((1,), (0,)), ((), ())),
            preferred_element_type=jnp.float32)
        scores = cb2_ref[i] - 2.0 * mm
        m = jnp.min(scores, axis=0, keepdims=True)
        idx = jnp.min(jnp.where(scores == m, iota0, K), axis=0, keepdims=True)
        onehot = (iota0 == idx).astype(jnp.bfloat16)
        q = (jax.lax.dot_general(hi_ref[i], onehot, (((1,), (0,)), ((), ())),
                                 preferred_element_type=jnp.float32)
             + jax.lax.dot_general(mid_ref[i], onehot, (((1,), (0,)), ((), ())),
                                   preferred_element_type=jnp.float32)) \
            + jax.lax.dot_general(lo_ref[i], onehot, (((1,), (0,)), ((), ())),
                                  preferred_element_type=jnp.float32)
        zqt_ref[i * cd:(i + 1) * cd, :] = q
        idx_ref[i:i + 1, :] = idx
        if i < n_cb - 1:
            carry = res - q


@functools.partial(jax.jit, static_argnames=())
def kernel(z, codebooks):
    B, T, D = z.shape
    n_cb, K, cd = codebooks.shape
    rows = B * T
    R = 512
    zt = z.reshape(rows, D).T
    cb_bf = codebooks.astype(jnp.bfloat16)
    cb2 = jnp.sum(codebooks * codebooks, axis=-1)[..., None]
    hi = jax.lax.optimization_barrier(codebooks.astype(jnp.bfloat16))
    r1 = codebooks - hi.astype(jnp.float32)
    mid = jax.lax.optimization_barrier(r1.astype(jnp.bfloat16))
    lo = (r1 - mid.astype(jnp.float32)).astype(jnp.bfloat16)
    hiT = jnp.swapaxes(hi, 1, 2)
    midT = jnp.swapaxes(mid, 1, 2)
    loT = jnp.swapaxes(lo, 1, 2)

    zqt, idx8 = pl.pallas_call(
        functools.partial(_rvq_block, n_cb=n_cb, K=K, cd=cd),
        grid=(rows // R,),
        in_specs=[
            pl.BlockSpec((D, R), lambda b: (0, b)),
            pl.BlockSpec((n_cb, K, cd), lambda b: (0, 0, 0)),
            pl.BlockSpec((n_cb, K, 1), lambda b: (0, 0, 0)),
            pl.BlockSpec((n_cb, cd, K), lambda b: (0, 0, 0)),
            pl.BlockSpec((n_cb, cd, K), lambda b: (0, 0, 0)),
            pl.BlockSpec((n_cb, cd, K), lambda b: (0, 0, 0)),
        ],
        out_specs=[
            pl.BlockSpec((D, R), lambda b: (0, b)),
            pl.BlockSpec((n_cb, R), lambda b: (0, b)),
        ],
        out_shape=[
            jax.ShapeDtypeStruct((D, rows), jnp.float32),
            jax.ShapeDtypeStruct((n_cb, rows), jnp.int32),
        ],
    )(zt, cb_bf, cb2, hiT, midT, loT)

    z_q = zqt.T.reshape(B, T, D)
    indices = idx8.reshape(n_cb, B, T).transpose(1, 0, 2)
    return (z_q, indices)
